# baseline (speedup 1.0000x reference)
import functools

import jax
import jax.numpy as jnp
from jax import lax
from jax.experimental import pallas as pl
from jax.experimental.pallas import tpu as pltpu
from jax.experimental.pallas import tpu_sc as plsc

N = 10000
E = 320000
D = 128
EPS = 1e-07

NC = 2
NS = 16
NW = NC * NS
EPW = E // NW
C = 80
NCHUNK = EPW // C
NGROUP = C // 16

_LN2 = 0.6931471805599453


def _ln16(x):
  bits = lax.bitcast_convert_type(x, jnp.int32)
  e = (bits >> 23) - 127
  m = lax.bitcast_convert_type(
      (bits & 0x007FFFFF) | 0x3F800000, jnp.float32)
  big = m >= 1.4142135623730951
  m = jnp.where(big, m * 0.5, m)
  e = e + jnp.where(big, 1, 0)
  t = m - 1.0
  s = t / (2.0 + t)
  s2 = s * s
  p = 2.0 / 9.0
  p = p * s2 + 2.0 / 7.0
  p = p * s2 + 2.0 / 5.0
  p = p * s2 + 2.0 / 3.0
  p = p * s2 + 2.0
  return e.astype(jnp.float32) * _LN2 + s * p


def _prep_body(z_ref, w_ref, b_ref, mz_ref, zp_ref):
  zb = z_ref[...]
  mz_ref[...] = jnp.sum(zb * w_ref[...], axis=1) + b_ref[0]
  zu = lax.bitcast_convert_type(zb, jnp.uint32)
  hi = zu[:, : D // 2] & jnp.uint32(0xFFFF0000)
  lo = zu[:, D // 2:] >> 16
  zp_ref[...] = lax.bitcast_convert_type(hi | lo, jnp.int32)


def _prep(z, W, b):
  return pl.pallas_call(
      _prep_body,
      out_shape=(
          jax.ShapeDtypeStruct((N,), jnp.float32),
          jax.ShapeDtypeStruct((N, D // 2), jnp.int32),
      ),
  )(z, W, b)


def _sc_body(z_hbm, ei_hbm, mz_hbm,
             lo_hbm, pr_hbm, mj_hbm, d2_hbm,
             mz_v, sidx, didx, sr0, sr1, dr0, dr1,
             lo_s, pr_s, mj_s, d2_s,
             ss0, ss1, ds0, ds1):
  wid = lax.axis_index("s") * NC + lax.axis_index("c")
  base = wid * EPW
  pltpu.sync_copy(mz_hbm, mz_v)
  pltpu.sync_copy(ei_hbm.at[0, pl.ds(base, EPW)], sidx)
  pltpu.sync_copy(ei_hbm.at[1, pl.ds(base, EPW)], didx)

  def issue(i, sbuf, dbuf, ssem, dsem):
    pltpu.async_copy(z_hbm.at[sidx.at[pl.ds(i * C, C)]], sbuf, ssem)
    pltpu.async_copy(z_hbm.at[didx.at[pl.ds(i * C, C)]], dbuf, dsem)

  def wait(i, sbuf, dbuf, ssem, dsem):
    pltpu.make_async_copy(
        z_hbm.at[sidx.at[pl.ds(i * C, C)]], sbuf, ssem).wait()
    pltpu.make_async_copy(
        z_hbm.at[didx.at[pl.ds(i * C, C)]], dbuf, dsem).wait()

  def compute(i, srows, drows):
    obase = i * C

    def group(g):
      lane = lax.iota(jnp.int32, 16)
      d2v = jnp.zeros((16,), jnp.float32)
      for j in range(16):
        e = g * 16 + j
        acc1 = jnp.zeros((16,), jnp.float32)
        acc2 = jnp.zeros((16,), jnp.float32)
        for k in range(D // 32):
          sl = pl.ds(k * 16, 16)
          a = plsc.bitcast(srows[e, sl], jnp.bfloat16)
          bb = plsc.bitcast(drows[e, sl], jnp.bfloat16)
          diff = a - bb
          sq = diff * diff
          bits = plsc.bitcast(sq, jnp.int32)
          acc1 = acc1 + lax.bitcast_convert_type(bits, jnp.float32)
          acc2 = acc2 + lax.bitcast_convert_type(bits << 16, jnp.float32)
        d2 = jnp.sum(acc1 + acc2)
        d2v = jnp.where(lane == j, d2, d2v)
      dist2 = d2v + EPS
      mjv = plsc.load_gather(mz_v, [didx[pl.ds(i * C + g * 16, 16)]])
      logits = mjv - _ln16(dist2)
      prob = 1.0 / (1.0 + jnp.exp(-logits))
      sl = pl.ds(obase + g * 16, 16)
      lo_s[sl] = logits
      pr_s[sl] = prob
      mj_s[sl] = mjv
      d2_s[sl] = dist2

    for g in range(NGROUP):
      group(g)

  issue(0, sr0, dr0, ss0, ds0)

  def pair(ih, _):
    i0 = ih * 2
    issue(i0 + 1, sr1, dr1, ss1, ds1)
    wait(i0, sr0, dr0, ss0, ds0)
    compute(i0, sr0, dr0)
    issue(i0 + 2, sr0, dr0, ss0, ds0)
    wait(i0 + 1, sr1, dr1, ss1, ds1)
    compute(i0 + 1, sr1, dr1)
    return 0

  lax.fori_loop(0, NCHUNK // 2, pair, 0)
  wait(NCHUNK - 1, sr0, dr0, ss0, ds0)
  compute(NCHUNK - 1, sr0, dr0)

  pltpu.sync_copy(lo_s, lo_hbm.at[pl.ds(base, EPW)])
  pltpu.sync_copy(pr_s, pr_hbm.at[pl.ds(base, EPW)])
  pltpu.sync_copy(mj_s, mj_hbm.at[pl.ds(base, EPW)])
  pltpu.sync_copy(d2_s, d2_hbm.at[pl.ds(base, EPW)])


_sc_call = functools.partial(
    pl.kernel,
    mesh=plsc.VectorSubcoreMesh(core_axis_name="c", subcore_axis_name="s"),
    compiler_params=pltpu.CompilerParams(
        needs_layout_passes=False, use_tc_tiling_on_sc=False),
    out_type=[jax.ShapeDtypeStruct((E,), jnp.float32)] * 4,
    scratch_types=[
        pltpu.VMEM((N,), jnp.float32),
        pltpu.VMEM((EPW,), jnp.int32),
        pltpu.VMEM((EPW,), jnp.int32),
        pltpu.VMEM((C, D // 2), jnp.int32),
        pltpu.VMEM((C, D // 2), jnp.int32),
        pltpu.VMEM((C, D // 2), jnp.int32),
        pltpu.VMEM((C, D // 2), jnp.int32),
        pltpu.VMEM((EPW,), jnp.float32),
        pltpu.VMEM((EPW,), jnp.float32),
        pltpu.VMEM((EPW,), jnp.float32),
        pltpu.VMEM((EPW,), jnp.float32),
        pltpu.SemaphoreType.DMA,
        pltpu.SemaphoreType.DMA,
        pltpu.SemaphoreType.DMA,
        pltpu.SemaphoreType.DMA,
    ],
)(_sc_body)


def kernel(z, edge_index, W, b):
  mz, zpack = _prep(z, W, b)
  logits, prob, mj, dist2 = _sc_call(zpack, edge_index, mz)
  return (logits, prob, mj, dist2)


# speedup vs baseline: 1.5361x; 1.5361x over previous
import functools

import jax
import jax.numpy as jnp
from jax import lax
from jax.experimental import pallas as pl
from jax.experimental.pallas import tpu as pltpu
from jax.experimental.pallas import tpu_sc as plsc

N = 10000
E = 320000
D = 128
EPS = 1e-07

NC = 2
NS = 16
NW = NC * NS
EPW = E // NW
C = 80
NCHUNK = EPW // C
NGROUP = C // 16

_LN2 = 0.6931471805599453


def _ln16(x):
  bits = lax.bitcast_convert_type(x, jnp.int32)
  e = (bits >> 23) - 127
  m = lax.bitcast_convert_type(
      (bits & 0x007FFFFF) | 0x3F800000, jnp.float32)
  big = m >= 1.4142135623730951
  m = jnp.where(big, m * 0.5, m)
  e = e + jnp.where(big, 1, 0)
  t = m - 1.0
  s = t / (2.0 + t)
  s2 = s * s
  p = 2.0 / 9.0
  p = p * s2 + 2.0 / 7.0
  p = p * s2 + 2.0 / 5.0
  p = p * s2 + 2.0 / 3.0
  p = p * s2 + 2.0
  return e.astype(jnp.float32) * _LN2 + s * p


def _prep_body(z_ref, w_ref, b_ref, mz_ref, zp_ref):
  zb = z_ref[...]
  mz_ref[...] = jnp.sum(zb * w_ref[...], axis=1) + b_ref[0]
  zu = lax.bitcast_convert_type(zb, jnp.uint32)
  hi = zu[:, : D // 2] & jnp.uint32(0xFFFF0000)
  lo = zu[:, D // 2:] >> 16
  zp_ref[...] = lax.bitcast_convert_type(hi | lo, jnp.int32)


def _prep(z, W, b):
  return pl.pallas_call(
      _prep_body,
      out_shape=(
          jax.ShapeDtypeStruct((N,), jnp.float32),
          jax.ShapeDtypeStruct((N, D // 2), jnp.int32),
      ),
  )(z, W, b)


def _sc_body(z_hbm, ei_hbm, mz_hbm,
             lo_hbm, pr_hbm, mj_hbm, d2_hbm,
             mz_v, sidx, didx, sr0, sr1, dr0, dr1,
             lo_s, pr_s, mj_s, d2_s,
             ss0, ss1, ds0, ds1):
  wid = lax.axis_index("s") * NC + lax.axis_index("c")
  base = wid * EPW
  pltpu.sync_copy(mz_hbm, mz_v)
  pltpu.sync_copy(ei_hbm.at[0, pl.ds(base, EPW)], sidx)
  pltpu.sync_copy(ei_hbm.at[1, pl.ds(base, EPW)], didx)

  def issue(i, sbuf, dbuf, ssem, dsem):
    pltpu.async_copy(z_hbm.at[sidx.at[pl.ds(i * C, C)]], sbuf, ssem)
    pltpu.async_copy(z_hbm.at[didx.at[pl.ds(i * C, C)]], dbuf, dsem)

  def wait(i, sbuf, dbuf, ssem, dsem):
    pltpu.make_async_copy(
        z_hbm.at[sidx.at[pl.ds(i * C, C)]], sbuf, ssem).wait()
    pltpu.make_async_copy(
        z_hbm.at[didx.at[pl.ds(i * C, C)]], dbuf, dsem).wait()

  def compute(i, srows, drows):
    obase = i * C

    def group(g, _):
      lane = lax.iota(jnp.int32, 16)
      d2v = jnp.zeros((16,), jnp.float32)
      for j in range(16):
        e = g * 16 + j
        acc1 = jnp.zeros((16,), jnp.float32)
        acc2 = jnp.zeros((16,), jnp.float32)
        for k in range(D // 32):
          sl = pl.ds(k * 16, 16)
          a = plsc.bitcast(srows[e, sl], jnp.bfloat16)
          bb = plsc.bitcast(drows[e, sl], jnp.bfloat16)
          diff = a - bb
          sq = diff * diff
          bits = plsc.bitcast(sq, jnp.int32)
          acc1 = acc1 + lax.bitcast_convert_type(bits, jnp.float32)
          acc2 = acc2 + lax.bitcast_convert_type(bits << 16, jnp.float32)
        d2 = jnp.sum(acc1 + acc2)
        d2v = jnp.where(lane == j, d2, d2v)
      dist2 = d2v + EPS
      mjv = plsc.load_gather(mz_v, [didx[pl.ds(i * C + g * 16, 16)]])
      logits = mjv - _ln16(dist2)
      prob = 1.0 / (1.0 + jnp.exp(-logits))
      sl = pl.ds(obase + g * 16, 16)
      lo_s[sl] = logits
      pr_s[sl] = prob
      mj_s[sl] = mjv
      d2_s[sl] = dist2
      return 0

    lax.fori_loop(0, NGROUP, group, 0)

  issue(0, sr0, dr0, ss0, ds0)

  def pair(ih, _):
    i0 = ih * 2
    issue(i0 + 1, sr1, dr1, ss1, ds1)
    wait(i0, sr0, dr0, ss0, ds0)
    compute(i0, sr0, dr0)
    issue(i0 + 2, sr0, dr0, ss0, ds0)
    wait(i0 + 1, sr1, dr1, ss1, ds1)
    compute(i0 + 1, sr1, dr1)
    return 0

  lax.fori_loop(0, NCHUNK // 2, pair, 0)
  wait(NCHUNK - 1, sr0, dr0, ss0, ds0)
  compute(NCHUNK - 1, sr0, dr0)

  pltpu.sync_copy(lo_s, lo_hbm.at[pl.ds(base, EPW)])
  pltpu.sync_copy(pr_s, pr_hbm.at[pl.ds(base, EPW)])
  pltpu.sync_copy(mj_s, mj_hbm.at[pl.ds(base, EPW)])
  pltpu.sync_copy(d2_s, d2_hbm.at[pl.ds(base, EPW)])


_sc_call = functools.partial(
    pl.kernel,
    mesh=plsc.VectorSubcoreMesh(core_axis_name="c", subcore_axis_name="s"),
    compiler_params=pltpu.CompilerParams(
        needs_layout_passes=False, use_tc_tiling_on_sc=False),
    out_type=[jax.ShapeDtypeStruct((E,), jnp.float32)] * 4,
    scratch_types=[
        pltpu.VMEM((N,), jnp.float32),
        pltpu.VMEM((EPW,), jnp.int32),
        pltpu.VMEM((EPW,), jnp.int32),
        pltpu.VMEM((C, D // 2), jnp.int32),
        pltpu.VMEM((C, D // 2), jnp.int32),
        pltpu.VMEM((C, D // 2), jnp.int32),
        pltpu.VMEM((C, D // 2), jnp.int32),
        pltpu.VMEM((EPW,), jnp.float32),
        pltpu.VMEM((EPW,), jnp.float32),
        pltpu.VMEM((EPW,), jnp.float32),
        pltpu.VMEM((EPW,), jnp.float32),
        pltpu.SemaphoreType.DMA,
        pltpu.SemaphoreType.DMA,
        pltpu.SemaphoreType.DMA,
        pltpu.SemaphoreType.DMA,
    ],
)(_sc_body)


def kernel(z, edge_index, W, b):
  mz, zpack = _prep(z, W, b)
  logits, prob, mj, dist2 = _sc_call(zpack, edge_index, mz)
  return (logits, prob, mj, dist2)


# speedup vs baseline: 1.7085x; 1.1122x over previous
import functools

import jax
import jax.numpy as jnp
from jax import lax
from jax.experimental import pallas as pl
from jax.experimental.pallas import tpu as pltpu
from jax.experimental.pallas import tpu_sc as plsc

N = 10000
E = 320000
D = 128
EPS = 1e-07

NC = 2
NS = 16
NW = NC * NS
EPW = E // NW
C = 80
NCHUNK = EPW // C
NGROUP = C // 16

_LN2 = 0.6931471805599453


def _ln16(x):
  bits = lax.bitcast_convert_type(x, jnp.int32)
  e = (bits >> 23) - 127
  m = lax.bitcast_convert_type(
      (bits & 0x007FFFFF) | 0x3F800000, jnp.float32)
  big = m >= 1.4142135623730951
  m = jnp.where(big, m * 0.5, m)
  e = e + jnp.where(big, 1, 0)
  t = m - 1.0
  s = t / (2.0 + t)
  s2 = s * s
  p = 2.0 / 9.0
  p = p * s2 + 2.0 / 7.0
  p = p * s2 + 2.0 / 5.0
  p = p * s2 + 2.0 / 3.0
  p = p * s2 + 2.0
  return e.astype(jnp.float32) * _LN2 + s * p


def _prep_body(z_ref, w_ref, b_ref, mz_ref, zp_ref):
  zb = z_ref[...]
  mz_ref[...] = jnp.sum(zb * w_ref[...], axis=1) + b_ref[0]
  zu = lax.bitcast_convert_type(zb, jnp.uint32)
  hi = zu[:, : D // 2] & jnp.uint32(0xFFFF0000)
  lo = zu[:, D // 2:] >> 16
  zp_ref[...] = lax.bitcast_convert_type(hi | lo, jnp.int32)


def _prep(z, W, b):
  return pl.pallas_call(
      _prep_body,
      out_shape=(
          jax.ShapeDtypeStruct((N,), jnp.float32),
          jax.ShapeDtypeStruct((N, D // 2), jnp.int32),
      ),
  )(z, W, b)


def _sc_body(z_hbm, ei_hbm, mz_hbm,
             lo_hbm, pr_hbm, mj_hbm, d2_hbm,
             mz_v, z_sh, sidx, didx, sr0, sr1, dr0, dr1,
             lo_s, pr_s, mj_s, d2_s,
             ss0, ss1, ds0, ds1):
  wid = lax.axis_index("s") * NC + lax.axis_index("c")
  sid = lax.axis_index("s")
  base = wid * EPW
  pltpu.sync_copy(mz_hbm, mz_v)
  pltpu.sync_copy(ei_hbm.at[0, pl.ds(base, EPW)], sidx)
  pltpu.sync_copy(ei_hbm.at[1, pl.ds(base, EPW)], didx)
  rows_per_sub = N // NS
  pltpu.sync_copy(z_hbm.at[pl.ds(sid * rows_per_sub, rows_per_sub)],
                  z_sh.at[pl.ds(sid * rows_per_sub, rows_per_sub)])
  plsc.subcore_barrier()

  def issue(i, sbuf, dbuf, ssem, dsem):
    pltpu.async_copy(z_sh.at[sidx.at[pl.ds(i * C, C)]], sbuf, ssem)
    pltpu.async_copy(z_sh.at[didx.at[pl.ds(i * C, C)]], dbuf, dsem)

  def wait(i, sbuf, dbuf, ssem, dsem):
    pltpu.make_async_copy(
        z_sh.at[sidx.at[pl.ds(i * C, C)]], sbuf, ssem).wait()
    pltpu.make_async_copy(
        z_sh.at[didx.at[pl.ds(i * C, C)]], dbuf, dsem).wait()

  def compute(i, srows, drows):
    obase = i * C

    def group(g, _):
      lane = lax.iota(jnp.int32, 16)
      d2v = jnp.zeros((16,), jnp.float32)
      for j in range(16):
        e = g * 16 + j
        acc1 = jnp.zeros((16,), jnp.float32)
        acc2 = jnp.zeros((16,), jnp.float32)
        for k in range(D // 32):
          sl = pl.ds(k * 16, 16)
          a = plsc.bitcast(srows[e, sl], jnp.bfloat16)
          bb = plsc.bitcast(drows[e, sl], jnp.bfloat16)
          diff = a - bb
          sq = diff * diff
          bits = plsc.bitcast(sq, jnp.int32)
          acc1 = acc1 + lax.bitcast_convert_type(bits, jnp.float32)
          acc2 = acc2 + lax.bitcast_convert_type(bits << 16, jnp.float32)
        d2 = jnp.sum(acc1 + acc2)
        d2v = jnp.where(lane == j, d2, d2v)
      dist2 = d2v + EPS
      mjv = plsc.load_gather(mz_v, [didx[pl.ds(i * C + g * 16, 16)]])
      logits = mjv - _ln16(dist2)
      prob = 1.0 / (1.0 + jnp.exp(-logits))
      sl = pl.ds(obase + g * 16, 16)
      lo_s[sl] = logits
      pr_s[sl] = prob
      mj_s[sl] = mjv
      d2_s[sl] = dist2
      return 0

    lax.fori_loop(0, NGROUP, group, 0)

  issue(0, sr0, dr0, ss0, ds0)

  def pair(ih, _):
    i0 = ih * 2
    issue(i0 + 1, sr1, dr1, ss1, ds1)
    wait(i0, sr0, dr0, ss0, ds0)
    compute(i0, sr0, dr0)
    issue(i0 + 2, sr0, dr0, ss0, ds0)
    wait(i0 + 1, sr1, dr1, ss1, ds1)
    compute(i0 + 1, sr1, dr1)
    return 0

  lax.fori_loop(0, NCHUNK // 2, pair, 0)
  wait(NCHUNK - 1, sr0, dr0, ss0, ds0)
  compute(NCHUNK - 1, sr0, dr0)

  pltpu.sync_copy(lo_s, lo_hbm.at[pl.ds(base, EPW)])
  pltpu.sync_copy(pr_s, pr_hbm.at[pl.ds(base, EPW)])
  pltpu.sync_copy(mj_s, mj_hbm.at[pl.ds(base, EPW)])
  pltpu.sync_copy(d2_s, d2_hbm.at[pl.ds(base, EPW)])


_sc_call = functools.partial(
    pl.kernel,
    mesh=plsc.VectorSubcoreMesh(core_axis_name="c", subcore_axis_name="s"),
    compiler_params=pltpu.CompilerParams(
        needs_layout_passes=False, use_tc_tiling_on_sc=False),
    out_type=[jax.ShapeDtypeStruct((E,), jnp.float32)] * 4,
    scratch_types=[
        pltpu.VMEM((N,), jnp.float32),
        pltpu.VMEM_SHARED((N, D // 2), jnp.int32),
        pltpu.VMEM((EPW,), jnp.int32),
        pltpu.VMEM((EPW,), jnp.int32),
        pltpu.VMEM((C, D // 2), jnp.int32),
        pltpu.VMEM((C, D // 2), jnp.int32),
        pltpu.VMEM((C, D // 2), jnp.int32),
        pltpu.VMEM((C, D // 2), jnp.int32),
        pltpu.VMEM((EPW,), jnp.float32),
        pltpu.VMEM((EPW,), jnp.float32),
        pltpu.VMEM((EPW,), jnp.float32),
        pltpu.VMEM((EPW,), jnp.float32),
        pltpu.SemaphoreType.DMA,
        pltpu.SemaphoreType.DMA,
        pltpu.SemaphoreType.DMA,
        pltpu.SemaphoreType.DMA,
    ],
)(_sc_body)


def kernel(z, edge_index, W, b):
  mz, zpack = _prep(z, W, b)
  logits, prob, mj, dist2 = _sc_call(zpack, edge_index, mz)
  return (logits, prob, mj, dist2)
